# baseline (speedup 1.0000x reference)
import jax
import jax.numpy as jnp
from jax import lax
from jax.experimental import pallas as pl
from jax.experimental.pallas import tpu as pltpu
from jax.experimental.pallas import tpu_sc as plsc

B = 16384
D = 128
HALF = 64
NC = 2
NS = 16
L = 16
NW = NC * NS
NSC = 8192
BPW = NSC // NW
GROUPS = BPW // L
MT_WORDS = 12 * HALF
HT_WORDS = 24 * HALF
TBL_WORDS = MT_WORDS + HT_WORDS
TBLK = 512


def _emb_body(mt_hbm, ht_hbm, months_hbm, hours_hbm, out_hbm,
              tbl_v, m_v, h_v, rows_v, sem):
    wid = lax.axis_index("s") * NC + lax.axis_index("c")
    base = wid * BPW
    copies = [
        pltpu.async_copy(mt_hbm, tbl_v.at[pl.ds(0, MT_WORDS)], sem),
        pltpu.async_copy(ht_hbm, tbl_v.at[pl.ds(MT_WORDS, HT_WORDS)], sem),
        pltpu.async_copy(months_hbm.at[pl.ds(base, BPW)], m_v, sem),
        pltpu.async_copy(hours_hbm.at[pl.ds(base, BPW)], h_v, sem),
    ]
    for c in copies:
        c.wait()

    lane = lax.iota(jnp.int32, L)
    lanec = [lane + c for c in range(0, HALF, L)]
    spl = [jnp.full((L,), l, jnp.int32) for l in range(L)]

    PIPE = 12

    def build(g):
        mm = m_v[pl.ds(g * L, L)]
        hh = h_v[pl.ds(g * L, L)]
        mb = mm * HALF
        hb = hh * HALF + MT_WORDS
        gbase = g * (L * D)
        pend = []

        def drain():
            off, v = pend.pop(0)
            rows_v[pl.ds(off, L)] = v

        for l in range(L):
            bm = mb.at[spl[l]].get(mode="promise_in_bounds")
            bh = hb.at[spl[l]].get(mode="promise_in_bounds")
            for ci, c in enumerate(range(0, HALF, L)):
                pend.append((gbase + l * D + c,
                             plsc.load_gather(tbl_v, [bm + lanec[ci]])))
                pend.append((gbase + l * D + HALF + c,
                             plsc.load_gather(tbl_v, [bh + lanec[ci]])))
                while len(pend) > PIPE:
                    drain()
        while pend:
            drain()

    NCHUNK = 8
    CG = GROUPS // NCHUNK
    CW = BPW * D // NCHUNK

    def _chunk_copy(c):
        return pltpu.make_async_copy(
            rows_v.at[pl.ds(c * CW, CW)],
            out_hbm.at[pl.ds(base * D + c * CW, CW)], sem)

    @pl.loop(0, GROUPS)
    def _(g):
        build(g)
        for c in range(NCHUNK - 1):
            @pl.when(g == (c + 1) * CG - 1)
            def _(c=c):
                _chunk_copy(c).start()

    pltpu.sync_copy(
        rows_v.at[pl.ds((NCHUNK - 1) * CW, CW)],
        out_hbm.at[pl.ds(base * D + (NCHUNK - 1) * CW, CW)])
    for c in range(NCHUNK - 1):
        _chunk_copy(c).wait()


def _tc_body(m_ref, h_ref, mt_ref, ht_ref, o_ref):
    m = m_ref[0, 0, :]
    h = h_ref[0, 0, :]
    mt = mt_ref[...]
    ht = ht_ref[...]
    mt_hi = mt.astype(jnp.bfloat16)
    mt_lo = (mt - mt_hi.astype(jnp.float32)).astype(jnp.bfloat16)
    ht_hi = ht.astype(jnp.bfloat16)
    ht_lo = (ht - ht_hi.astype(jnp.float32)).astype(jnp.bfloat16)
    ohm = (m[:, None] == lax.broadcasted_iota(jnp.int32, (1, 12), 1)).astype(jnp.bfloat16)
    ohh = (h[:, None] == lax.broadcasted_iota(jnp.int32, (1, 24), 1)).astype(jnp.bfloat16)
    om = (jnp.dot(ohm, mt_hi, preferred_element_type=jnp.float32)
          + jnp.dot(ohm, mt_lo, preferred_element_type=jnp.float32))
    oh = (jnp.dot(ohh, ht_hi, preferred_element_type=jnp.float32)
          + jnp.dot(ohh, ht_lo, preferred_element_type=jnp.float32))
    o_ref[...] = jnp.concatenate([om, oh], axis=1)


def _tc_part(months, hours, mt, ht):
    n = months.shape[0]
    m3 = months.reshape(n // TBLK, 1, TBLK)
    h3 = hours.reshape(n // TBLK, 1, TBLK)
    return pl.pallas_call(
        _tc_body,
        grid=(n // TBLK,),
        in_specs=[
            pl.BlockSpec((1, 1, TBLK), lambda i: (i, 0, 0)),
            pl.BlockSpec((1, 1, TBLK), lambda i: (i, 0, 0)),
            pl.BlockSpec((12, 64), lambda i: (0, 0)),
            pl.BlockSpec((24, 64), lambda i: (0, 0)),
        ],
        out_specs=pl.BlockSpec((TBLK, D), lambda i: (i, 0)),
        out_shape=jax.ShapeDtypeStruct((n, D), jnp.float32),
    )(m3, h3, mt, ht)


def kernel(months, hours, month_table, hour_table):
    months = months.astype(jnp.int32)
    hours = hours.astype(jnp.int32)
    mesh = plsc.VectorSubcoreMesh(core_axis_name="c", subcore_axis_name="s")
    cp = pltpu.CompilerParams(needs_layout_passes=False, use_tc_tiling_on_sc=False,
                              disable_bounds_checks=True,
                              disable_semaphore_checks=True)
    run = pl.kernel(
        _emb_body,
        out_type=jax.ShapeDtypeStruct((NSC * D,), jnp.float32),
        mesh=mesh,
        scratch_types=[
            pltpu.VMEM((TBL_WORDS,), jnp.float32),
            pltpu.VMEM((BPW,), jnp.int32),
            pltpu.VMEM((BPW,), jnp.int32),
            pltpu.VMEM((BPW * D,), jnp.float32),
            pltpu.SemaphoreType.DMA,
        ],
        compiler_params=cp,
    )
    sc_out = run(month_table.reshape(-1), hour_table.reshape(-1),
                 months[:NSC], hours[:NSC]).reshape(NSC, D)
    tc_out = _tc_part(months[NSC:], hours[NSC:], month_table, hour_table)
    return jnp.concatenate([sc_out, tc_out], axis=0)


# speedup vs baseline: 1.5818x; 1.5818x over previous
import jax
import jax.numpy as jnp
from jax import lax
from jax.experimental import pallas as pl
from jax.experimental.pallas import tpu as pltpu
from jax.experimental.pallas import tpu_sc as plsc

B = 16384
D = 128
HALF = 64
NC = 2
NS = 16
L = 16
NW = NC * NS
BPW = B // NW
GROUPS = BPW // L


def _emb_body(mt_hbm, ht_hbm, months_hbm, hours_hbm, out_hbm,
              mt_v, ht_v, m_v, h_v, rows_v, sem):
    wid = lax.axis_index("s") * NC + lax.axis_index("c")
    base = wid * BPW
    copies = [
        pltpu.async_copy(mt_hbm, mt_v, sem),
        pltpu.async_copy(ht_hbm, ht_v, sem),
        pltpu.async_copy(months_hbm.at[pl.ds(base, BPW)], m_v, sem),
        pltpu.async_copy(hours_hbm.at[pl.ds(base, BPW)], h_v, sem),
    ]
    for c in copies:
        c.wait()

    lane = lax.iota(jnp.int32, L)
    lanec = [lane + c for c in range(0, HALF, L)]
    spl = [jnp.full((L,), l, jnp.int32) for l in range(L)]

    PIPE = 12

    def build(g):
        mm = m_v[pl.ds(g * L, L)]
        hh = h_v[pl.ds(g * L, L)]
        gbase = g * (L * D)
        pend = []

        def drain():
            off, v = pend.pop(0)
            rows_v[pl.ds(off, L)] = v

        for l in range(L):
            bm = mm.at[spl[l]].get(mode="promise_in_bounds")
            bh = hh.at[spl[l]].get(mode="promise_in_bounds")
            for ci, c in enumerate(range(0, HALF, L)):
                pend.append((gbase + l * D + c,
                             plsc.load_gather(mt_v, [bm, lanec[ci]])))
                pend.append((gbase + l * D + HALF + c,
                             plsc.load_gather(ht_v, [bh, lanec[ci]])))
                while len(pend) > PIPE:
                    drain()
        while pend:
            drain()

    NCHUNK = 8
    CG = GROUPS // NCHUNK
    CW = BPW * D // NCHUNK

    def _chunk_copy(c):
        return pltpu.make_async_copy(
            rows_v.at[pl.ds(c * CW, CW)],
            out_hbm.at[pl.ds(base * D + c * CW, CW)], sem)

    @pl.loop(0, GROUPS)
    def _(g):
        build(g)
        for c in range(NCHUNK - 1):
            @pl.when(g == (c + 1) * CG - 1)
            def _(c=c):
                _chunk_copy(c).start()

    pltpu.sync_copy(
        rows_v.at[pl.ds((NCHUNK - 1) * CW, CW)],
        out_hbm.at[pl.ds(base * D + (NCHUNK - 1) * CW, CW)])
    for c in range(NCHUNK - 1):
        _chunk_copy(c).wait()


def kernel(months, hours, month_table, hour_table):
    mesh = plsc.VectorSubcoreMesh(core_axis_name="c", subcore_axis_name="s")
    cp = pltpu.CompilerParams(needs_layout_passes=False, use_tc_tiling_on_sc=False,
                              disable_bounds_checks=True,
                              disable_semaphore_checks=True)
    run = pl.kernel(
        _emb_body,
        out_type=jax.ShapeDtypeStruct((B * D,), jnp.float32),
        mesh=mesh,
        scratch_types=[
            pltpu.VMEM((12, HALF), jnp.float32),
            pltpu.VMEM((24, HALF), jnp.float32),
            pltpu.VMEM((BPW,), jnp.int32),
            pltpu.VMEM((BPW,), jnp.int32),
            pltpu.VMEM((BPW * D,), jnp.float32),
            pltpu.SemaphoreType.DMA,
        ],
        compiler_params=cp,
    )
    out = run(month_table, hour_table,
              months.astype(jnp.int32), hours.astype(jnp.int32))
    return out.reshape(B, D)


# speedup vs baseline: 1.5954x; 1.0086x over previous
import jax
import jax.numpy as jnp
from jax import lax
from jax.experimental import pallas as pl
from jax.experimental.pallas import tpu as pltpu
from jax.experimental.pallas import tpu_sc as plsc

B = 16384
D = 128
HALF = 64
NC = 2
NS = 16
L = 16
NW = NC * NS
BPW = B // NW
GROUPS = BPW // L
MT_WORDS = 12 * HALF
TBL_WORDS = MT_WORDS + 24 * HALF


def _emb_body(tbl_hbm, months_hbm, hours_hbm, out_hbm,
              tbl_v, m_v, h_v, rows_v, sem):
    wid = lax.axis_index("s") * NC + lax.axis_index("c")
    base = wid * BPW
    copies = [
        pltpu.async_copy(tbl_hbm, tbl_v, sem),
        pltpu.async_copy(months_hbm.at[pl.ds(base, BPW)], m_v, sem),
        pltpu.async_copy(hours_hbm.at[pl.ds(base, BPW)], h_v, sem),
    ]
    for c in copies:
        c.wait()

    lane = lax.iota(jnp.int32, L)
    lanec = [lane + c for c in range(0, HALF, L)]
    spl = [jnp.full((L,), l, jnp.int32) for l in range(L)]

    PIPE = 12

    def build(g):
        mb = m_v[pl.ds(g * L, L)] * HALF
        hb = h_v[pl.ds(g * L, L)] * HALF + MT_WORDS
        gbase = g * (L * D)
        pend = []

        def drain():
            off, v = pend.pop(0)
            rows_v[pl.ds(off, L)] = v

        for l in range(L):
            bm = mb.at[spl[l]].get(mode="promise_in_bounds")
            bh = hb.at[spl[l]].get(mode="promise_in_bounds")
            for ci, c in enumerate(range(0, HALF, L)):
                pend.append((gbase + l * D + c,
                             plsc.load_gather(tbl_v, [bm + lanec[ci]])))
                pend.append((gbase + l * D + HALF + c,
                             plsc.load_gather(tbl_v, [bh + lanec[ci]])))
                while len(pend) > PIPE:
                    drain()
        while pend:
            drain()

    NCHUNK = 8
    CG = GROUPS // NCHUNK
    CW = BPW * D // NCHUNK

    def _chunk_copy(c):
        return pltpu.make_async_copy(
            rows_v.at[pl.ds(c * CW, CW)],
            out_hbm.at[pl.ds(base * D + c * CW, CW)], sem)

    @pl.loop(0, GROUPS)
    def _(g):
        build(g)
        for c in range(NCHUNK - 1):
            @pl.when(g == (c + 1) * CG - 1)
            def _(c=c):
                _chunk_copy(c).start()

    pltpu.sync_copy(
        rows_v.at[pl.ds((NCHUNK - 1) * CW, CW)],
        out_hbm.at[pl.ds(base * D + (NCHUNK - 1) * CW, CW)])
    for c in range(NCHUNK - 1):
        _chunk_copy(c).wait()


def kernel(months, hours, month_table, hour_table):
    mesh = plsc.VectorSubcoreMesh(core_axis_name="c", subcore_axis_name="s")
    cp = pltpu.CompilerParams(needs_layout_passes=False, use_tc_tiling_on_sc=False,
                              disable_bounds_checks=True,
                              disable_semaphore_checks=True)
    run = pl.kernel(
        _emb_body,
        out_type=jax.ShapeDtypeStruct((B * D,), jnp.float32),
        mesh=mesh,
        scratch_types=[
            pltpu.VMEM((TBL_WORDS,), jnp.float32),
            pltpu.VMEM((BPW,), jnp.int32),
            pltpu.VMEM((BPW,), jnp.int32),
            pltpu.VMEM((BPW * D,), jnp.float32),
            pltpu.SemaphoreType.DMA,
        ],
        compiler_params=cp,
    )
    tbl = jnp.concatenate([month_table.reshape(-1), hour_table.reshape(-1)])
    out = run(tbl, months.astype(jnp.int32), hours.astype(jnp.int32))
    return out.reshape(B, D)
